# baseline (speedup 1.0000x reference)
import functools

import jax
import jax.numpy as jnp
from jax import lax
from jax.experimental import pallas as pl
from jax.experimental.pallas import tpu as pltpu
from jax.experimental.pallas import tpu_sc as plsc

_BATCH = 1024
_BLOCK = 50
_VOCAB = 100000
_EMB = 32
_HID = 512

_NC, _NS = 2, 16
_NW = _NC * _NS
_ROWS = _BATCH * _BLOCK
_R_PER_W = _ROWS // _NW
_CHUNK = 80
_NCHUNK = _R_PER_W // _CHUNK

_TILE_V = 2048
_NFULL = _VOCAB // _TILE_V
_TAIL = _VOCAB - _NFULL * _TILE_V
_TAIL_ALIGNED = (_TAIL // 128) * 128
_SLIVER_IDX = (_NFULL * _TILE_V + _TAIL_ALIGNED) // 128
_GRID = _NFULL + 1
_NBUF = 4


def _sc_gather(emb, idx3):
    mesh = plsc.VectorSubcoreMesh(core_axis_name="c", subcore_axis_name="s")

    @functools.partial(
        pl.kernel,
        out_type=jax.ShapeDtypeStruct((_ROWS, _EMB), jnp.float32),
        mesh=mesh,
        scratch_types=[
            pltpu.VMEM((_NCHUNK, _CHUNK), jnp.int32),
            pltpu.VMEM((_R_PER_W, _EMB), jnp.float32),
            pltpu.SemaphoreType.DMA,
        ],
        compiler_params=pltpu.CompilerParams(use_tc_tiling_on_sc=False),
    )
    def gather_kernel(table_hbm, idx_hbm, out_hbm, idx_v, rows_v, sem):
        wid = lax.axis_index("s") * _NC + lax.axis_index("c")
        base = wid * _R_PER_W
        pltpu.sync_copy(idx_hbm.at[wid], idx_v)
        descs = [
            pltpu.make_async_copy(
                table_hbm.at[idx_v.at[j]],
                rows_v.at[pl.ds(j * _CHUNK, _CHUNK)],
                sem,
            )
            for j in range(_NCHUNK)
        ]
        for d in descs:
            d.start()
        for d in descs:
            d.wait()
        pltpu.sync_copy(rows_v, out_hbm.at[pl.ds(base, _R_PER_W)])

    return gather_kernel(emb, idx3)


def _tc_mlp(e, W1, b1, W2, b2):
    def body(e_ref, w1_ref, b1_ref, w2_ref, b2_ref, o_hbm, h_ref, o_bufs, sems):
        i = pl.program_id(0)
        slot = lax.rem(i, _NBUF)

        @pl.when(i == 0)
        def _():
            h = lax.dot_general(
                e_ref[...], w1_ref[...],
                (((1,), (1,)), ((), ())),
                preferred_element_type=jnp.float32,
            )
            h_ref[...] = jnp.maximum(h + b1_ref[...], 0.0)

        @pl.when(i >= _NBUF)
        def _():
            pltpu.make_async_copy(
                o_bufs.at[slot],
                o_hbm.at[:, pl.ds((i - _NBUF) * _TILE_V, _TILE_V)],
                sems.at[slot],
            ).wait()

        o_bufs[slot] = lax.dot_general(
            h_ref[...], w2_ref[...],
            (((1,), (1,)), ((), ())),
            preferred_element_type=jnp.float32,
        ) + b2_ref[...]

        @pl.when(i < _NFULL)
        def _():
            pltpu.make_async_copy(
                o_bufs.at[slot],
                o_hbm.at[:, pl.ds(i * _TILE_V, _TILE_V)],
                sems.at[slot],
            ).start()

        @pl.when(i == _NFULL)
        def _():
            tail = pltpu.make_async_copy(
                o_bufs.at[slot, :, pl.ds(0, _TAIL_ALIGNED)],
                o_hbm.at[:, pl.ds(_NFULL * _TILE_V, _TAIL_ALIGNED)],
                sems.at[slot],
            )
            tail.start()
            tail.wait()
            for k in range(_NFULL - _NBUF + 1, _NFULL):
                s = k % _NBUF
                pltpu.make_async_copy(
                    o_bufs.at[s],
                    o_hbm.at[:, pl.ds(k * _TILE_V, _TILE_V)],
                    sems.at[s],
                ).wait()

    return pl.pallas_call(
        body,
        grid=(_GRID,),
        in_specs=[
            pl.BlockSpec((_BATCH, _BLOCK * _EMB), lambda i: (0, 0)),
            pl.BlockSpec((_HID, _BLOCK * _EMB), lambda i: (0, 0)),
            pl.BlockSpec((1, _HID), lambda i: (0, 0)),
            pl.BlockSpec((_TILE_V, _HID), lambda i: (i, 0)),
            pl.BlockSpec((1, _TILE_V), lambda i: (0, i)),
        ],
        out_specs=[
            pl.BlockSpec(memory_space=pltpu.HBM),
            pl.BlockSpec((_BATCH, _HID), lambda i: (0, 0)),
        ],
        out_shape=[
            jax.ShapeDtypeStruct((_BATCH, _VOCAB), jnp.float32),
            jax.ShapeDtypeStruct((_BATCH, _HID), jnp.float32),
        ],
        scratch_shapes=[
            pltpu.VMEM((_NBUF, _BATCH, _TILE_V), jnp.float32),
            pltpu.SemaphoreType.DMA((_NBUF,)),
        ],
        compiler_params=pltpu.CompilerParams(
            vmem_limit_bytes=112 * 1024 * 1024,
        ),
    )(e, W1, b1.reshape(1, _HID), W2, b2.reshape(1, _VOCAB))


def _sliver(out_main, h, W2, b2):
    def body(h_ref, w2_ref, b2_ref, _, o_ref):
        o_ref[...] = lax.dot_general(
            h_ref[...], w2_ref[...],
            (((1,), (1,)), ((), ())),
            preferred_element_type=jnp.float32,
        ) + b2_ref[...]

    return pl.pallas_call(
        body,
        grid=(1,),
        in_specs=[
            pl.BlockSpec((_BATCH, _HID), lambda i: (0, 0)),
            pl.BlockSpec((128, _HID), lambda i: (_SLIVER_IDX, 0)),
            pl.BlockSpec((1, 128), lambda i: (0, _SLIVER_IDX)),
            pl.BlockSpec(memory_space=pltpu.HBM),
        ],
        out_specs=pl.BlockSpec((_BATCH, 128), lambda i: (0, _SLIVER_IDX)),
        out_shape=jax.ShapeDtypeStruct((_BATCH, _VOCAB), jnp.float32),
        input_output_aliases={3: 0},
    )(h, W2, b2.reshape(1, _VOCAB), out_main)


def kernel(x, emb, W1, b1, W2, b2):
    idx3 = x.astype(jnp.int32).reshape(_NW, _NCHUNK, _CHUNK)
    e = _sc_gather(emb, idx3).reshape(_BATCH, _BLOCK * _EMB)
    out_main, h = _tc_mlp(e, W1, b1, W2, b2)
    return _sliver(out_main, h, W2, b2)


# speedup vs baseline: 2.2442x; 2.2442x over previous
import functools

import jax
import jax.numpy as jnp
from jax import lax
from jax.experimental import pallas as pl
from jax.experimental.pallas import tpu as pltpu
from jax.experimental.pallas import tpu_sc as plsc

_BATCH = 1024
_BLOCK = 50
_VOCAB = 100000
_EMB = 32
_HID = 512

_NC, _NS = 2, 16
_NW = _NC * _NS
_ROWS = _BATCH * _BLOCK
_R_PER_W = _ROWS // _NW
_CHUNK = 80
_NCHUNK = _R_PER_W // _CHUNK

_TILE_V = 2048
_NFULL = _VOCAB // _TILE_V
_TAIL = _VOCAB - _NFULL * _TILE_V
_GRID = _NFULL + 1
_NBUF = 4


def _sc_gather(emb, idx3):
    mesh = plsc.VectorSubcoreMesh(core_axis_name="c", subcore_axis_name="s")

    @functools.partial(
        pl.kernel,
        out_type=jax.ShapeDtypeStruct((_ROWS, _EMB), jnp.float32),
        mesh=mesh,
        scratch_types=[
            pltpu.VMEM((_NCHUNK, _CHUNK), jnp.int32),
            pltpu.VMEM((_R_PER_W, _EMB), jnp.float32),
            pltpu.SemaphoreType.DMA,
        ],
        compiler_params=pltpu.CompilerParams(use_tc_tiling_on_sc=False),
    )
    def gather_kernel(table_hbm, idx_hbm, out_hbm, idx_v, rows_v, sem):
        wid = lax.axis_index("s") * _NC + lax.axis_index("c")
        base = wid * _R_PER_W
        pltpu.sync_copy(idx_hbm.at[wid], idx_v)
        descs = [
            pltpu.make_async_copy(
                table_hbm.at[idx_v.at[j]],
                rows_v.at[pl.ds(j * _CHUNK, _CHUNK)],
                sem,
            )
            for j in range(_NCHUNK)
        ]
        for d in descs:
            d.start()
        for d in descs:
            d.wait()
        pltpu.sync_copy(rows_v, out_hbm.at[pl.ds(base, _R_PER_W)])

    return gather_kernel(emb, idx3)


def _tc_mlp(e, W1, b1, W2, b2):
    def body(e_ref, w1_ref, b1_ref, w2_ref, b2_ref, o_hbm, h_ref, o_bufs, sems):
        i = pl.program_id(0)
        slot = lax.rem(i, _NBUF)

        @pl.when(i == 0)
        def _():
            h = lax.dot_general(
                e_ref[...], w1_ref[...],
                (((1,), (1,)), ((), ())),
                preferred_element_type=jnp.float32,
            )
            h_ref[...] = jnp.maximum(h + b1_ref[...], 0.0)

        @pl.when(i >= _NBUF)
        def _():
            pltpu.make_async_copy(
                o_bufs.at[slot],
                o_hbm.at[pl.ds((i - _NBUF) * _TILE_V, _TILE_V)],
                sems.at[slot],
            ).wait()

        o_bufs[slot] = lax.dot_general(
            w2_ref[...], h_ref[...],
            (((1,), (1,)), ((), ())),
            preferred_element_type=jnp.float32,
        ) + jnp.transpose(b2_ref[...])

        @pl.when(i < _NFULL)
        def _():
            pltpu.make_async_copy(
                o_bufs.at[slot],
                o_hbm.at[pl.ds(i * _TILE_V, _TILE_V)],
                sems.at[slot],
            ).start()

        @pl.when(i == _NFULL)
        def _():
            tail = pltpu.make_async_copy(
                o_bufs.at[slot, pl.ds(0, _TAIL)],
                o_hbm.at[pl.ds(_NFULL * _TILE_V, _TAIL)],
                sems.at[slot],
            )
            tail.start()
            tail.wait()
            for k in range(_NFULL - _NBUF + 1, _NFULL):
                s = k % _NBUF
                pltpu.make_async_copy(
                    o_bufs.at[s],
                    o_hbm.at[pl.ds(k * _TILE_V, _TILE_V)],
                    sems.at[s],
                ).wait()

    return pl.pallas_call(
        body,
        grid=(_GRID,),
        in_specs=[
            pl.BlockSpec((_BATCH, _BLOCK * _EMB), lambda i: (0, 0)),
            pl.BlockSpec((_HID, _BLOCK * _EMB), lambda i: (0, 0)),
            pl.BlockSpec((1, _HID), lambda i: (0, 0)),
            pl.BlockSpec((_TILE_V, _HID), lambda i: (i, 0)),
            pl.BlockSpec((1, _TILE_V), lambda i: (0, i)),
        ],
        out_specs=pl.BlockSpec(memory_space=pltpu.HBM),
        out_shape=jax.ShapeDtypeStruct((_VOCAB, _BATCH), jnp.float32),
        scratch_shapes=[
            pltpu.VMEM((_BATCH, _HID), jnp.float32),
            pltpu.VMEM((_NBUF, _TILE_V, _BATCH), jnp.float32),
            pltpu.SemaphoreType.DMA((_NBUF,)),
        ],
        compiler_params=pltpu.CompilerParams(
            vmem_limit_bytes=112 * 1024 * 1024,
        ),
    )(e, W1, b1.reshape(1, _HID), W2, b2.reshape(1, _VOCAB))


def kernel(x, emb, W1, b1, W2, b2):
    idx3 = x.astype(jnp.int32).reshape(_NW, _NCHUNK, _CHUNK)
    e = _sc_gather(emb, idx3).reshape(_BATCH, _BLOCK * _EMB)
    out_t = _tc_mlp(e, W1, b1, W2, b2)
    return jnp.transpose(out_t)
